# baseline (speedup 1.0000x reference)
import functools

import jax
import jax.numpy as jnp
from jax import lax
from jax.experimental import pallas as pl
from jax.experimental.pallas import tpu as pltpu
from jax.experimental.pallas import tpu_sc as plsc

N_NODES = 10000
D = 128
N_EDGES = 320000

NC = 2
NS = 16
NW = NC * NS

CH = 128
N0 = 160
assert N0 % 2 == 0
N_CHUNKS = NS * N0
EDGES_PAD = N_CHUNKS * CH
ACC_ROWS = 10240
ROWS_PER_TILE = ACC_ROWS // NS


def _linear_body(feat_ref, on_ref, w_ref, y_ref):
    x = feat_ref[...] / on_ref[...]
    y_ref[...] = lax.dot_general(
        x, w_ref[...], (((1,), (1,)), ((), ())),
        preferred_element_type=jnp.float32,
        precision=lax.Precision.HIGHEST,
    )


def _linear(feat, out_norm2d, W):
    blk = 1000
    return pl.pallas_call(
        _linear_body,
        grid=(N_NODES // blk,),
        in_specs=[
            pl.BlockSpec((blk, D), lambda i: (i, 0)),
            pl.BlockSpec((blk, 1), lambda i: (i, 0)),
            pl.BlockSpec((D, D), lambda i: (0, 0)),
        ],
        out_specs=pl.BlockSpec((blk, D), lambda i: (i, 0)),
        out_shape=jax.ShapeDtypeStruct((N_NODES, D), jnp.float32),
    )(feat, out_norm2d, W)


@functools.partial(
    pl.kernel,
    out_type=jax.ShapeDtypeStruct((ACC_ROWS, D), jnp.float32),
    mesh=plsc.VectorSubcoreMesh(core_axis_name="c", subcore_axis_name="s"),
    scratch_types=[
        pltpu.VMEM((CH,), jnp.int32),
        pltpu.VMEM((CH,), jnp.int32),
        pltpu.VMEM((CH,), jnp.int32),
        pltpu.VMEM((CH,), jnp.int32),
        pltpu.VMEM((CH, D), jnp.float32),
        pltpu.VMEM((CH, D), jnp.float32),
        pltpu.VMEM_SHARED((ACC_ROWS, D), jnp.float32),
        pltpu.SemaphoreType.DMA,
        pltpu.SemaphoreType.DMA,
        pltpu.SemaphoreType.DMA,
        pltpu.SemaphoreType.DMA,
    ],
)
def _sc_aggregate(y_hbm, src_hbm, dst_hbm, out_hbm,
                  sidx0, sidx1, didx0, didx1, rows0, rows1, acc,
                  semi0, semi1, semg0, semg1):
    cid = lax.axis_index("c")
    sid = lax.axis_index("s")
    sidx = (sidx0, sidx1)
    didx = (didx0, didx1)
    semi = (semi0, semi1)
    rows = (rows0, rows1)
    semg = (semg0, semg1)

    base = sid * N0
    n = N0

    def _stage_idx(c, b):
        pltpu.async_copy(src_hbm.at[base + c], sidx[b], semi[b])
        pltpu.async_copy(dst_hbm.at[base + c], didx[b], semi[b])

    def _wait_idx(b):
        pltpu.make_async_copy(src_hbm.at[base], sidx[b], semi[b]).wait()
        pltpu.make_async_copy(dst_hbm.at[base], didx[b], semi[b]).wait()

    def _fire_gather(c, b):
        pltpu.async_copy(y_hbm.at[sidx[b]], rows[b], semg[b])

    def _wait_gather(b):
        pltpu.make_async_copy(y_hbm.at[sidx[b]], rows[b], semg[b]).wait()

    @pl.when(cid == 0)
    def _prologue():
        _stage_idx(0, 0)
        _stage_idx(1, 1)

        def _zrow(i, carry):
            for t in range(D // 16):
                rows0[i, pl.ds(t * 16, 16)] = jnp.zeros((16,), jnp.float32)
            return carry

        lax.fori_loop(0, CH, _zrow, 0)
        for r in range(ROWS_PER_TILE // CH):
            pltpu.sync_copy(rows0, acc.at[pl.ds(sid * ROWS_PER_TILE + r * CH, CH)])

        _wait_idx(0)
        _fire_gather(0, 0)

    plsc.subcore_barrier()

    def _step(i, carry):
        b = lax.rem(i, 2)
        nb = 1 - b
        cn = jnp.minimum(i + 1, n - 1)

        @pl.when(nb == 0)
        def _():
            _wait_idx(0)
            _fire_gather(cn, 0)

        @pl.when(nb == 1)
        def _():
            _wait_idx(1)
            _fire_gather(cn, 1)

        @pl.when(b == 0)
        def _():
            _wait_gather(0)
            pltpu.sync_copy(rows0, acc.at[didx0], add=True)
            _stage_idx(jnp.minimum(i + 2, n - 1), 0)

        @pl.when(b == 1)
        def _():
            _wait_gather(1)
            pltpu.sync_copy(rows1, acc.at[didx1], add=True)
            _stage_idx(jnp.minimum(i + 2, n - 1), 1)

        return carry

    @pl.when(cid == 0)
    def _main():
        lax.fori_loop(0, n, _step, 0)
        _wait_gather(0)
        _wait_idx(1)

    plsc.subcore_barrier()

    @pl.when(cid == 0)
    def _writeback():
        pltpu.sync_copy(
            acc.at[pl.ds(sid * ROWS_PER_TILE, ROWS_PER_TILE)],
            out_hbm.at[pl.ds(sid * ROWS_PER_TILE, ROWS_PER_TILE)],
        )


def _combine_body(p_ref, inn_ref, b_ref, o_ref):
    o_ref[...] = p_ref[...] / inn_ref[...] + b_ref[...]


def _combine(partial, in_norm2d, b2d):
    blk = 2000
    return pl.pallas_call(
        _combine_body,
        grid=(N_NODES // blk,),
        in_specs=[
            pl.BlockSpec((blk, D), lambda i: (i, 0)),
            pl.BlockSpec((blk, 1), lambda i: (i, 0)),
            pl.BlockSpec((1, D), lambda i: (0, 0)),
        ],
        out_specs=pl.BlockSpec((blk, D), lambda i: (i, 0)),
        out_shape=jax.ShapeDtypeStruct((N_NODES, D), jnp.float32),
    )(partial, in_norm2d, b2d)


def kernel(feat, in_norm, out_norm, edge_index, W, b):
    y = _linear(feat, out_norm.reshape(N_NODES, 1), W)

    src = edge_index[0].astype(jnp.int32)
    dst = edge_index[1].astype(jnp.int32)
    pad = EDGES_PAD - N_EDGES
    src3 = jnp.concatenate([src, jnp.zeros((pad,), jnp.int32)]).reshape(N_CHUNKS, CH)
    dst3 = jnp.concatenate([dst, jnp.full((pad,), N_NODES, jnp.int32)]).reshape(N_CHUNKS, CH)

    partial = _sc_aggregate(y, src3, dst3)
    return _combine(partial, in_norm.reshape(N_NODES, 1), b.reshape(1, D))


# speedup vs baseline: 2.9766x; 2.9766x over previous
import functools

import jax
import jax.numpy as jnp
from jax import lax
from jax.experimental import pallas as pl
from jax.experimental.pallas import tpu as pltpu
from jax.experimental.pallas import tpu_sc as plsc

N_NODES = 10000
D = 128
N_EDGES = 320000

NC = 2
NS = 16
NW = NC * NS

CH = 128
N_CHUNKS = N_EDGES // CH
NB0 = 78
CORE0_TOTAL = NS * NB0 + 4
ACC_ROWS = 10240
ROWS_PER_TILE = ACC_ROWS // NS


def _linear_body(feat_ref, on_ref, w_ref, y_ref):
    x = feat_ref[...] / on_ref[...]
    y_ref[...] = lax.dot_general(
        x, w_ref[...], (((1,), (1,)), ((), ())),
        preferred_element_type=jnp.float32,
        precision=lax.Precision.HIGHEST,
    )


def _linear(feat, out_norm2d, W):
    blk = 1000
    return pl.pallas_call(
        _linear_body,
        grid=(N_NODES // blk,),
        in_specs=[
            pl.BlockSpec((blk, D), lambda i: (i, 0)),
            pl.BlockSpec((blk, 1), lambda i: (i, 0)),
            pl.BlockSpec((D, D), lambda i: (0, 0)),
        ],
        out_specs=pl.BlockSpec((blk, D), lambda i: (i, 0)),
        out_shape=jax.ShapeDtypeStruct((N_NODES, D), jnp.float32),
    )(feat, out_norm2d, W)


@functools.partial(
    pl.kernel,
    out_type=jax.ShapeDtypeStruct((NC, ACC_ROWS, D), jnp.float32),
    mesh=plsc.VectorSubcoreMesh(core_axis_name="c", subcore_axis_name="s"),
    scratch_types=[
        pltpu.VMEM((CH,), jnp.int32),
        pltpu.VMEM((CH,), jnp.int32),
        pltpu.VMEM((CH,), jnp.int32),
        pltpu.VMEM((CH,), jnp.int32),
        pltpu.VMEM((CH, D), jnp.float32),
        pltpu.VMEM((CH, D), jnp.float32),
        pltpu.VMEM_SHARED((ACC_ROWS, D), jnp.float32),
        pltpu.SemaphoreType.DMA,
        pltpu.SemaphoreType.DMA,
        pltpu.SemaphoreType.DMA,
        pltpu.SemaphoreType.DMA,
    ],
)
def _sc_aggregate(y_hbm, src_hbm, dst_hbm, out_hbm,
                  sidx0, sidx1, didx0, didx1, rows0, rows1, acc,
                  semi0, semi1, semg0, semg1):
    cid = lax.axis_index("c")
    sid = lax.axis_index("s")
    sidx = (sidx0, sidx1)
    didx = (didx0, didx1)
    semi = (semi0, semi1)
    rows = (rows0, rows1)
    semg = (semg0, semg1)

    base = jnp.where(cid == 0,
                     sid * NB0 + 2 * jnp.minimum(sid, 2),
                     CORE0_TOTAL + sid * NB0)
    n = jnp.where((cid == 0) & (sid < 2), NB0 + 2, NB0)

    def _stage_idx(c, b):
        pltpu.async_copy(src_hbm.at[base + c], sidx[b], semi[b])
        pltpu.async_copy(dst_hbm.at[base + c], didx[b], semi[b])

    def _wait_idx(b):
        pltpu.make_async_copy(src_hbm.at[base], sidx[b], semi[b]).wait()
        pltpu.make_async_copy(dst_hbm.at[base], didx[b], semi[b]).wait()

    def _fire_gather(c, b):
        pltpu.async_copy(y_hbm.at[sidx[b]], rows[b], semg[b])

    def _wait_gather(b):
        pltpu.make_async_copy(y_hbm.at[sidx[b]], rows[b], semg[b]).wait()

    _stage_idx(0, 0)
    _stage_idx(1, 1)

    def _zrow(i, carry):
        for t in range(D // 16):
            rows0[i, pl.ds(t * 16, 16)] = jnp.zeros((16,), jnp.float32)
        return carry

    lax.fori_loop(0, CH, _zrow, 0)
    for r in range(ROWS_PER_TILE // CH):
        pltpu.sync_copy(rows0, acc.at[pl.ds(sid * ROWS_PER_TILE + r * CH, CH)])

    _wait_idx(0)
    _fire_gather(0, 0)
    plsc.subcore_barrier()

    def _step(i, carry):
        b = lax.rem(i, 2)
        nb = 1 - b
        cn = jnp.minimum(i + 1, n - 1)

        @pl.when(nb == 0)
        def _():
            _wait_idx(0)
            _fire_gather(cn, 0)

        @pl.when(nb == 1)
        def _():
            _wait_idx(1)
            _fire_gather(cn, 1)

        @pl.when(b == 0)
        def _():
            _wait_gather(0)
            pltpu.sync_copy(rows0, acc.at[didx0], add=True)
            _stage_idx(jnp.minimum(i + 2, n - 1), 0)

        @pl.when(b == 1)
        def _():
            _wait_gather(1)
            pltpu.sync_copy(rows1, acc.at[didx1], add=True)
            _stage_idx(jnp.minimum(i + 2, n - 1), 1)

        return carry

    lax.fori_loop(0, n, _step, 0)
    _wait_gather(0)
    _wait_idx(1)
    plsc.subcore_barrier()

    pltpu.sync_copy(
        acc.at[pl.ds(sid * ROWS_PER_TILE, ROWS_PER_TILE)],
        out_hbm.at[cid, pl.ds(sid * ROWS_PER_TILE, ROWS_PER_TILE)],
    )


def _combine_body(p_ref, inn_ref, b_ref, o_ref):
    o_ref[...] = (p_ref[0] + p_ref[1]) / inn_ref[...] + b_ref[...]


def _combine(partial, in_norm2d, b2d):
    blk = 2000
    return pl.pallas_call(
        _combine_body,
        grid=(N_NODES // blk,),
        in_specs=[
            pl.BlockSpec((NC, blk, D), lambda i: (0, i, 0)),
            pl.BlockSpec((blk, 1), lambda i: (i, 0)),
            pl.BlockSpec((1, D), lambda i: (0, 0)),
        ],
        out_specs=pl.BlockSpec((blk, D), lambda i: (i, 0)),
        out_shape=jax.ShapeDtypeStruct((N_NODES, D), jnp.float32),
    )(partial, in_norm2d, b2d)


def kernel(feat, in_norm, out_norm, edge_index, W, b):
    y = _linear(feat, out_norm.reshape(N_NODES, 1), W)

    src3 = edge_index[0].astype(jnp.int32).reshape(N_CHUNKS, CH)
    dst3 = edge_index[1].astype(jnp.int32).reshape(N_CHUNKS, CH)

    partial = _sc_aggregate(y, src3, dst3)
    return _combine(partial, in_norm.reshape(N_NODES, 1), b.reshape(1, D))


# speedup vs baseline: 3.2741x; 1.0999x over previous
import functools

import jax
import jax.numpy as jnp
from jax import lax
from jax.experimental import pallas as pl
from jax.experimental.pallas import tpu as pltpu
from jax.experimental.pallas import tpu_sc as plsc

N_NODES = 10000
D = 128
N_EDGES = 320000

NC = 2
NS = 16
NW = NC * NS

CH = 128
N_CHUNKS = N_EDGES // CH
NB0 = 78
CORE0_TOTAL = NS * NB0 + 4
ACC_ROWS = 10240
ROWS_PER_TILE = ACC_ROWS // NS


def _linear_body(feat_ref, on_ref, w_ref, y_ref):
    x = feat_ref[...] / on_ref[...]
    y_ref[...] = lax.dot_general(
        x, w_ref[...], (((1,), (1,)), ((), ())),
        preferred_element_type=jnp.float32,
        precision=lax.Precision.HIGHEST,
    )


def _linear(feat, out_norm2d, W):
    blk = 1000
    return pl.pallas_call(
        _linear_body,
        grid=(N_NODES // blk,),
        in_specs=[
            pl.BlockSpec((blk, D), lambda i: (i, 0)),
            pl.BlockSpec((blk, 1), lambda i: (i, 0)),
            pl.BlockSpec((D, D), lambda i: (0, 0)),
        ],
        out_specs=pl.BlockSpec((blk, D), lambda i: (i, 0)),
        out_shape=jax.ShapeDtypeStruct((N_NODES, D), jnp.float32),
    )(feat, out_norm2d, W)


@functools.partial(
    pl.kernel,
    out_type=jax.ShapeDtypeStruct((NC, ACC_ROWS, D), jnp.float32),
    mesh=plsc.VectorSubcoreMesh(core_axis_name="c", subcore_axis_name="s"),
    scratch_types=[
        pltpu.VMEM((CH,), jnp.int32),
        pltpu.VMEM((CH,), jnp.int32),
        pltpu.VMEM((CH,), jnp.int32),
        pltpu.VMEM((CH,), jnp.int32),
        pltpu.VMEM((CH,), jnp.int32),
        pltpu.VMEM((CH,), jnp.int32),
        pltpu.VMEM((CH, D), jnp.float32),
        pltpu.VMEM((CH, D), jnp.float32),
        pltpu.VMEM_SHARED((ACC_ROWS, D), jnp.float32),
        pltpu.SemaphoreType.DMA,
        pltpu.SemaphoreType.DMA,
        pltpu.SemaphoreType.DMA,
        pltpu.SemaphoreType.DMA,
        pltpu.SemaphoreType.DMA,
        pltpu.SemaphoreType.DMA,
    ],
)
def _sc_aggregate(y_hbm, src_hbm, dst_hbm, out_hbm,
                  sidx0, sidx1, didx0, didx1, dscat0, dscat1, rows0, rows1, acc,
                  semi0, semi1, semg0, semg1, semsc0, semsc1):
    cid = lax.axis_index("c")
    sid = lax.axis_index("s")
    sidx = (sidx0, sidx1)
    didx = (didx0, didx1)
    dscat = (dscat0, dscat1)
    semi = (semi0, semi1)
    rows = (rows0, rows1)
    semg = (semg0, semg1)
    semsc = (semsc0, semsc1)

    base = jnp.where(cid == 0,
                     sid * NB0 + 2 * jnp.minimum(sid, 2),
                     CORE0_TOTAL + sid * NB0)
    n = jnp.where((cid == 0) & (sid < 2), NB0 + 2, NB0)

    def _stage_idx(c, b):
        pltpu.async_copy(src_hbm.at[base + c], sidx[b], semi[b])
        pltpu.async_copy(dst_hbm.at[base + c], didx[b], semi[b])

    def _wait_idx(b):
        pltpu.make_async_copy(src_hbm.at[base], sidx[b], semi[b]).wait()
        pltpu.make_async_copy(dst_hbm.at[base], didx[b], semi[b]).wait()

    def _fire_gather(c, b):
        pltpu.async_copy(y_hbm.at[sidx[b]], rows[b], semg[b])

    def _wait_gather(b):
        pltpu.make_async_copy(y_hbm.at[sidx[b]], rows[b], semg[b]).wait()

    def _fire_scatter(b):
        for t in range(CH // 16):
            dscat[b][pl.ds(t * 16, 16)] = didx[b][pl.ds(t * 16, 16)]
        pltpu.async_copy(rows[b], acc.at[dscat[b]], semsc[b], add=True)

    def _wait_scatter(b):
        pltpu.make_async_copy(rows[b], acc.at[dscat[b]], semsc[b]).wait()

    _stage_idx(0, 0)
    _stage_idx(1, 1)

    def _zrow(i, carry):
        for t in range(D // 16):
            rows0[i, pl.ds(t * 16, 16)] = jnp.zeros((16,), jnp.float32)
        return carry

    lax.fori_loop(0, CH, _zrow, 0)
    for r in range(ROWS_PER_TILE // CH):
        pltpu.sync_copy(rows0, acc.at[pl.ds(sid * ROWS_PER_TILE + r * CH, CH)])

    _wait_idx(0)
    _fire_gather(0, 0)
    plsc.subcore_barrier()

    def _step(i, carry):
        b = lax.rem(i, 2)
        nb = 1 - b
        cn = jnp.minimum(i + 1, n - 1)

        @pl.when(nb == 0)
        def _():
            _wait_scatter(0)
            _wait_idx(0)
            _fire_gather(cn, 0)

        @pl.when(nb == 1)
        def _():
            @pl.when(i > 0)
            def _():
                _wait_scatter(1)
            _wait_idx(1)
            _fire_gather(cn, 1)

        @pl.when(b == 0)
        def _():
            _wait_gather(0)
            _fire_scatter(0)
            _stage_idx(jnp.minimum(i + 2, n - 1), 0)

        @pl.when(b == 1)
        def _():
            _wait_gather(1)
            _fire_scatter(1)
            _stage_idx(jnp.minimum(i + 2, n - 1), 1)

        return carry

    lax.fori_loop(0, n, _step, 0)
    _wait_gather(0)
    _wait_idx(1)
    _wait_scatter(1)
    plsc.subcore_barrier()

    pltpu.sync_copy(
        acc.at[pl.ds(sid * ROWS_PER_TILE, ROWS_PER_TILE)],
        out_hbm.at[cid, pl.ds(sid * ROWS_PER_TILE, ROWS_PER_TILE)],
    )


def _combine_body(p_ref, inn_ref, b_ref, o_ref):
    o_ref[...] = (p_ref[0] + p_ref[1]) / inn_ref[...] + b_ref[...]


def _combine(partial, in_norm2d, b2d):
    blk = 2000
    return pl.pallas_call(
        _combine_body,
        grid=(N_NODES // blk,),
        in_specs=[
            pl.BlockSpec((NC, blk, D), lambda i: (0, i, 0)),
            pl.BlockSpec((blk, 1), lambda i: (i, 0)),
            pl.BlockSpec((1, D), lambda i: (0, 0)),
        ],
        out_specs=pl.BlockSpec((blk, D), lambda i: (i, 0)),
        out_shape=jax.ShapeDtypeStruct((N_NODES, D), jnp.float32),
    )(partial, in_norm2d, b2d)


def kernel(feat, in_norm, out_norm, edge_index, W, b):
    y = _linear(feat, out_norm.reshape(N_NODES, 1), W)

    src3 = edge_index[0].astype(jnp.int32).reshape(N_CHUNKS, CH)
    dst3 = edge_index[1].astype(jnp.int32).reshape(N_CHUNKS, CH)

    partial = _sc_aggregate(y, src3, dst3)
    return _combine(partial, in_norm.reshape(N_NODES, 1), b.reshape(1, D))


# speedup vs baseline: 3.6977x; 1.1294x over previous
import functools

import jax
import jax.numpy as jnp
from jax import lax
from jax.experimental import pallas as pl
from jax.experimental.pallas import tpu as pltpu
from jax.experimental.pallas import tpu_sc as plsc

N_NODES = 10000
D = 128
N_EDGES = 320000

NC = 2
NS = 16
NW = NC * NS

CH = 128
N_CHUNKS = N_EDGES // CH
NB0 = 78
CORE0_TOTAL = NS * NB0 + 4
ACC_ROWS = 10240
ROWS_PER_TILE = ACC_ROWS // NS


def _linear_body(feat_ref, on_ref, w_ref, y_ref):
    x = feat_ref[...] / on_ref[...]
    y_ref[...] = lax.dot_general(
        x, w_ref[...], (((1,), (1,)), ((), ())),
        preferred_element_type=jnp.float32,
        precision=lax.Precision.DEFAULT,
    )


def _linear(feat, out_norm2d, W):
    blk = 1000
    return pl.pallas_call(
        _linear_body,
        grid=(N_NODES // blk,),
        in_specs=[
            pl.BlockSpec((blk, D), lambda i: (i, 0)),
            pl.BlockSpec((blk, 1), lambda i: (i, 0)),
            pl.BlockSpec((D, D), lambda i: (0, 0)),
        ],
        out_specs=pl.BlockSpec((blk, D), lambda i: (i, 0)),
        out_shape=jax.ShapeDtypeStruct((N_NODES, D), jnp.float32),
    )(feat, out_norm2d, W)


@functools.partial(
    pl.kernel,
    out_type=jax.ShapeDtypeStruct((NC, ACC_ROWS, D), jnp.float32),
    mesh=plsc.VectorSubcoreMesh(core_axis_name="c", subcore_axis_name="s"),
    scratch_types=[
        pltpu.VMEM((CH,), jnp.int32),
        pltpu.VMEM((CH,), jnp.int32),
        pltpu.VMEM((CH,), jnp.int32),
        pltpu.VMEM((CH,), jnp.int32),
        pltpu.VMEM((CH,), jnp.int32),
        pltpu.VMEM((CH,), jnp.int32),
        pltpu.VMEM((CH, D), jnp.float32),
        pltpu.VMEM((CH, D), jnp.float32),
        pltpu.VMEM_SHARED((ACC_ROWS, D), jnp.float32),
        pltpu.SemaphoreType.DMA,
        pltpu.SemaphoreType.DMA,
        pltpu.SemaphoreType.DMA,
        pltpu.SemaphoreType.DMA,
        pltpu.SemaphoreType.DMA,
        pltpu.SemaphoreType.DMA,
    ],
)
def _sc_aggregate(y_hbm, ei_hbm, out_hbm,
                  sidx0, sidx1, didx0, didx1, dscat0, dscat1, rows0, rows1, acc,
                  semi0, semi1, semg0, semg1, semsc0, semsc1):
    cid = lax.axis_index("c")
    sid = lax.axis_index("s")
    sidx = (sidx0, sidx1)
    didx = (didx0, didx1)
    dscat = (dscat0, dscat1)
    semi = (semi0, semi1)
    rows = (rows0, rows1)
    semg = (semg0, semg1)
    semsc = (semsc0, semsc1)

    base = jnp.where(cid == 0,
                     sid * NB0 + 2 * jnp.minimum(sid, 2),
                     CORE0_TOTAL + sid * NB0)
    n = jnp.where((cid == 0) & (sid < 2), NB0 + 2, NB0)

    def _stage_idx(c, b):
        pltpu.async_copy(ei_hbm.at[0, pl.ds((base + c) * CH, CH)], sidx[b], semi[b])
        pltpu.async_copy(ei_hbm.at[1, pl.ds((base + c) * CH, CH)], didx[b], semi[b])

    def _wait_idx(b):
        pltpu.make_async_copy(ei_hbm.at[0, pl.ds(0, CH)], sidx[b], semi[b]).wait()
        pltpu.make_async_copy(ei_hbm.at[1, pl.ds(0, CH)], didx[b], semi[b]).wait()

    def _fire_gather(c, b):
        pltpu.async_copy(y_hbm.at[sidx[b]], rows[b], semg[b])

    def _wait_gather(b):
        pltpu.make_async_copy(y_hbm.at[sidx[b]], rows[b], semg[b]).wait()

    def _fire_scatter(b):
        for t in range(CH // 16):
            dscat[b][pl.ds(t * 16, 16)] = didx[b][pl.ds(t * 16, 16)]
        pltpu.async_copy(rows[b], acc.at[dscat[b]], semsc[b], add=True)

    def _wait_scatter(b):
        pltpu.make_async_copy(rows[b], acc.at[dscat[b]], semsc[b]).wait()

    _stage_idx(0, 0)
    _stage_idx(1, 1)

    def _zrow(i, carry):
        for t in range(D // 16):
            rows0[i, pl.ds(t * 16, 16)] = jnp.zeros((16,), jnp.float32)
        return carry

    lax.fori_loop(0, CH, _zrow, 0)
    for r in range(ROWS_PER_TILE // CH):
        pltpu.sync_copy(rows0, acc.at[pl.ds(sid * ROWS_PER_TILE + r * CH, CH)])

    _wait_idx(0)
    _fire_gather(0, 0)
    plsc.subcore_barrier()

    def _step(i, carry):
        b = lax.rem(i, 2)
        nb = 1 - b
        cn = jnp.minimum(i + 1, n - 1)

        @pl.when(nb == 0)
        def _():
            _wait_scatter(0)
            _wait_idx(0)
            _fire_gather(cn, 0)

        @pl.when(nb == 1)
        def _():
            @pl.when(i > 0)
            def _():
                _wait_scatter(1)
            _wait_idx(1)
            _fire_gather(cn, 1)

        @pl.when(b == 0)
        def _():
            _wait_gather(0)
            _fire_scatter(0)
            _stage_idx(jnp.minimum(i + 2, n - 1), 0)

        @pl.when(b == 1)
        def _():
            _wait_gather(1)
            _fire_scatter(1)
            _stage_idx(jnp.minimum(i + 2, n - 1), 1)

        return carry

    lax.fori_loop(0, n, _step, 0)
    _wait_gather(0)
    _wait_idx(1)
    _wait_scatter(1)
    plsc.subcore_barrier()

    pltpu.sync_copy(
        acc.at[pl.ds(sid * ROWS_PER_TILE, ROWS_PER_TILE)],
        out_hbm.at[cid, pl.ds(sid * ROWS_PER_TILE, ROWS_PER_TILE)],
    )


def _combine_body(p_ref, inn_ref, b_ref, o_ref):
    o_ref[...] = (p_ref[0] + p_ref[1]) / inn_ref[...] + b_ref[...]


def _combine(partial, in_norm2d, b2d):
    blk = 2000
    return pl.pallas_call(
        _combine_body,
        grid=(N_NODES // blk,),
        in_specs=[
            pl.BlockSpec((NC, blk, D), lambda i: (0, i, 0)),
            pl.BlockSpec((blk, 1), lambda i: (i, 0)),
            pl.BlockSpec((1, D), lambda i: (0, 0)),
        ],
        out_specs=pl.BlockSpec((blk, D), lambda i: (i, 0)),
        out_shape=jax.ShapeDtypeStruct((N_NODES, D), jnp.float32),
    )(partial, in_norm2d, b2d)


def kernel(feat, in_norm, out_norm, edge_index, W, b):
    y = _linear(feat, out_norm.reshape(N_NODES, 1), W)

    partial = _sc_aggregate(y, edge_index.astype(jnp.int32))
    return _combine(partial, in_norm.reshape(N_NODES, 1), b.reshape(1, D))
